# baseline (speedup 1.0000x reference)
import functools

import jax
import jax.numpy as jnp
from jax import lax
from jax.experimental import pallas as pl
from jax.experimental.pallas import tpu as pltpu
from jax.experimental.pallas import tpu_sc as plsc


def _make_lookup(B, V, D):
    info = plsc.get_sparse_core_info()
    num_workers = info.num_cores * info.num_subcores
    b_per_w = B // num_workers
    assert B % num_workers == 0
    mesh = plsc.VectorSubcoreMesh(core_axis_name="c", subcore_axis_name="s")

    @functools.partial(
        pl.kernel,
        mesh=mesh,
        out_type=jax.ShapeDtypeStruct((B, D), jnp.float32),
        scratch_types=[
            pltpu.VMEM((b_per_w,), jnp.int32),
            pltpu.VMEM((b_per_w, D), jnp.float32),
            pltpu.SemaphoreType.DMA,
        ],
        compiler_params=pltpu.CompilerParams(use_tc_tiling_on_sc=False),
    )
    def lookup(idx_hbm, table_hbm, out_hbm, idx_v, rows_v, sem):
        wid = lax.axis_index("s") * info.num_cores + lax.axis_index("c")
        base = wid * b_per_w
        pltpu.sync_copy(idx_hbm.at[pl.ds(base, b_per_w)], idx_v)
        pltpu.async_copy(table_hbm.at[idx_v], rows_v, sem).wait()
        pltpu.sync_copy(rows_v, out_hbm.at[pl.ds(base, b_per_w)])

    return lookup


def kernel(category, table):
    B, = category.shape
    V, D = table.shape
    lookup = _make_lookup(B, V, D)
    return lookup(category.astype(jnp.int32), table)


# speedup vs baseline: 4.1007x; 4.1007x over previous
import functools

import jax
import jax.numpy as jnp
from jax import lax
from jax.experimental import pallas as pl
from jax.experimental.pallas import tpu as pltpu
from jax.experimental.pallas import tpu_sc as plsc

_LANE = 128


def _make_lookup(B, V, D):
    info = plsc.get_sparse_core_info()
    L = info.num_lanes
    num_workers = info.num_cores * info.num_subcores
    cpw = B // num_workers
    assert B % num_workers == 0 and cpw % L == 0
    ngroups = cpw // L
    mesh = plsc.VectorSubcoreMesh(core_axis_name="c", subcore_axis_name="s")

    @functools.partial(
        pl.kernel,
        mesh=mesh,
        out_type=jax.ShapeDtypeStruct((D, B), jnp.float32),
        scratch_types=[
            pltpu.VMEM((cpw,), jnp.int32),
            pltpu.VMEM((L, D, _LANE), jnp.float32),
            pltpu.VMEM((D, cpw), jnp.float32),
            pltpu.SemaphoreType.DMA((L,)),
        ],
        compiler_params=pltpu.CompilerParams(needs_layout_passes=False),
    )
    def lookup(idx_hbm, tab_hbm, out_hbm, idx_v, ring_v, out_v, sems):
        wid = lax.axis_index("s") * info.num_cores + lax.axis_index("c")
        base = wid * cpw
        pltpu.sync_copy(idx_hbm.at[pl.ds(base, cpw)], idx_v)

        def start(c, slot):
            c0 = pl.multiple_of((c // _LANE) * _LANE, _LANE)
            pltpu.async_copy(
                tab_hbm.at[:, pl.ds(c0, _LANE)],
                ring_v.at[slot],
                sems.at[slot],
            )

        def finish(g, c, slot):
            pltpu.make_async_copy(
                tab_hbm.at[:, pl.ds(0, _LANE)],
                ring_v.at[slot],
                sems.at[slot],
            ).wait()
            lane = jnp.broadcast_to(c % _LANE, (L,))
            col = jnp.broadcast_to(g, (L,))
            for half in range(D // L):
                rows = lax.iota(jnp.int32, L) + half * L
                vals = plsc.load_gather(ring_v.at[slot], [rows, lane])
                plsc.store_scatter(out_v, [rows, col], vals)

        first = idx_v[pl.ds(0, L)]
        for b in range(L):
            start(first[b], b)

        def outer(i, prev):
            cur = idx_v[pl.ds(i * L, L)]
            for b in range(L):
                finish((i - 1) * L + b, prev[b], b)
                start(cur[b], b)
            return cur

        last = lax.fori_loop(1, ngroups, outer, first)
        for b in range(L):
            finish((ngroups - 1) * L + b, last[b], b)
        pltpu.sync_copy(out_v, out_hbm.at[:, pl.ds(base, cpw)])

    return lookup


def kernel(category, table):
    B, = category.shape
    V, D = table.shape
    lookup = _make_lookup(B, V, D)
    return lookup(category.astype(jnp.int32), table.T).T
